# Initial kernel scaffold; baseline (speedup 1.0000x reference)
import jax
import jax.numpy as jnp
from jax.experimental import pallas as pl


def kernel(node_attributes, edge_indices, W0, b0, gamma0, beta0, mm0, mv0, W1, b1, gamma1, beta1, mm1, mv1, W2, b2, gamma2, beta2, mm2, mv2, Wp1, bp1, Wp2, bp2):
    raise NotImplementedError("write your pallas kernel here")


# speedup vs baseline: 1.2731x; 1.2731x over previous
import functools
import jax
import jax.numpy as jnp
from jax import lax
from jax.experimental import pallas as pl
from jax.experimental.pallas import tpu as pltpu
from jax.experimental.pallas import tpu_sc as plsc

N, E, D, U = 10000, 320000, 128, 128
NC, NS = 2, 16
NW = NC * NS
NP = 10240
R = NP // NW
S = 6400
NSLAB = E // S
CPS = S // 16
BG = 128
STAG = S + 16

_mesh = plsc.VectorSubcoreMesh(core_axis_name="c", subcore_axis_name="s")


@functools.partial(
    pl.kernel,
    out_type=(
        jax.ShapeDtypeStruct((NP, D), jnp.float32),
        jax.ShapeDtypeStruct((NP, D), jnp.float32),
        jax.ShapeDtypeStruct((NP,), jnp.float32),
    ),
    mesh=_mesh,
    compiler_params=pltpu.CompilerParams(needs_layout_passes=False),
    scratch_types=[
        pltpu.VMEM((R, D), jnp.float32),
        pltpu.VMEM((R, D), jnp.float32),
        pltpu.VMEM((R,), jnp.float32),
        pltpu.VMEM((S,), jnp.int32),
        pltpu.VMEM((S,), jnp.int32),
        pltpu.VMEM((STAG,), jnp.int32),
        pltpu.VMEM((STAG,), jnp.int32),
        pltpu.VMEM((BG, D), jnp.float32),
        pltpu.SemaphoreType.DMA,
    ],
)
def _sc_segment(x_hbm, dst_hbm, src_hbm, sum_hbm, max_hbm, cnt_hbm,
                accs, accm, cntv, dstv, srcv, stag_r, stag_s, gbuf, gsem):
    wid = lax.axis_index("s") * NC + lax.axis_index("c")
    row_base = wid * R

    zeros16 = jnp.zeros((16,), jnp.float32)
    neginf16 = jnp.full((16,), -jnp.inf, jnp.float32)
    zi16 = jnp.zeros((16,), jnp.int32)

    def init_row(r, _):
        for k in range(D // 16):
            accs[r, pl.ds(k * 16, 16)] = zeros16
            accm[r, pl.ds(k * 16, 16)] = neginf16
        return 0
    lax.fori_loop(0, R, init_row, 0)

    def init_cnt(i, _):
        cntv[pl.ds(i * 16, 16)] = zeros16
        return 0
    lax.fori_loop(0, R // 16, init_cnt, 0)

    def init_stag(i, _):
        stag_s[pl.ds(i * 16, 16)] = zi16
        return 0
    lax.fori_loop(0, STAG // 16, init_stag, 0)

    ones16 = jnp.ones((16,), jnp.float32)

    def slab_body(si, _):
        pltpu.sync_copy(dst_hbm.at[pl.ds(si * S, S)], dstv)
        pltpu.sync_copy(src_hbm.at[pl.ds(si * S, S)], srcv)

        def chunk_body(c, fill):
            d16 = dstv[pl.ds(c * 16, 16)]
            s16 = srcv[pl.ds(c * 16, 16)]
            loc = d16 - row_base
            mask = jnp.logical_and(loc >= 0, loc < R)
            locc = jnp.where(mask, loc, 0)
            plsc.addupdate_scatter(cntv, [locc], ones16, mask=mask)
            plsc.store_compressed(stag_r.at[pl.ds(fill, 16)], loc, mask=mask)
            plsc.store_compressed(stag_s.at[pl.ds(fill, 16)], s16, mask=mask)
            return fill + jnp.sum(mask.astype(jnp.int32))

        fill = lax.fori_loop(0, CPS, chunk_body, 0)
        nb = (fill + (BG - 1)) // BG

        def batch_body(bi, _):
            base = bi * BG
            pltpu.async_copy(x_hbm.at[stag_s.at[pl.ds(base, BG)]], gbuf,
                             gsem).wait()
            ec = jnp.minimum(fill - base, BG)

            def edge_body(j, _):
                r = stag_r[pl.ds(base + j, 16)][0]
                for k in range(D // 16):
                    sl = pl.ds(k * 16, 16)
                    g = gbuf[j, sl]
                    accs[r, sl] = accs[r, sl] + g
                    accm[r, sl] = jnp.maximum(accm[r, sl], g)
                return 0
            lax.fori_loop(0, ec, edge_body, 0)
            return 0
        lax.fori_loop(0, nb, batch_body, 0)
        return 0

    lax.fori_loop(0, NSLAB, slab_body, 0)

    pltpu.sync_copy(accs, sum_hbm.at[pl.ds(row_base, R)])
    pltpu.sync_copy(accm, max_hbm.at[pl.ds(row_base, R)])
    pltpu.sync_copy(cntv, cnt_hbm.at[pl.ds(row_base, R)])


def _layer_body(s_ref, m_ref, c_ref, wa_ref, wb_ref, b_ref, mm_ref, mv_ref,
                ga_ref, be_ref, o_ref):
    s = s_ref[...]
    mx = m_ref[...]
    c = c_ref[...]
    deg = jnp.maximum(c, 1.0)
    inv = 1.0 / deg
    scale = jnp.log(deg + 1.0) * (1.0 / 2.302585092994046)
    mean = s * inv
    mxf = jnp.where(c == 0.0, 0.0, mx)
    feats = jnp.concatenate([mean, mxf, s], axis=1)
    a = jnp.dot(feats, wa_ref[...], preferred_element_type=jnp.float32,
                precision=lax.Precision.HIGHEST)
    b = jnp.dot(feats, wb_ref[...], preferred_element_type=jnp.float32,
                precision=lax.Precision.HIGHEST)
    h = jnp.maximum(a + scale * b + b_ref[...], 0.0)
    h = (h - mm_ref[...]) / jnp.sqrt(mv_ref[...] + 1e-3) * ga_ref[...] \
        + be_ref[...]
    o_ref[...] = h


_BR = 256


def _tc_layer(ssum, smax, cnt2d, wa, wb, b2, mm2, mv2, ga2, be2):
    full = lambda shp: pl.BlockSpec(shp, lambda i: (0, 0))
    return pl.pallas_call(
        _layer_body,
        grid=(NP // _BR,),
        in_specs=[
            pl.BlockSpec((_BR, D), lambda i: (i, 0)),
            pl.BlockSpec((_BR, D), lambda i: (i, 0)),
            pl.BlockSpec((_BR, 1), lambda i: (i, 0)),
            full((3 * D, U)), full((3 * D, U)),
            full((1, U)), full((1, U)), full((1, U)), full((1, U)),
            full((1, U)),
        ],
        out_specs=pl.BlockSpec((_BR, U), lambda i: (i, 0)),
        out_shape=jax.ShapeDtypeStruct((NP, U), jnp.float32),
    )(ssum, smax, cnt2d, wa, wb, b2, mm2, mv2, ga2, be2)


def _proj_body(h_ref, w1_ref, b1_ref, w2_ref, b2_ref, o_ref):
    h = h_ref[...]
    rid = lax.broadcasted_iota(jnp.int32, (NP, U), 0)
    g = jnp.sum(jnp.where(rid < N, h, 0.0), axis=0, keepdims=True) / N
    g1 = jnp.maximum(jnp.dot(g, w1_ref[...],
                             preferred_element_type=jnp.float32,
                precision=lax.Precision.HIGHEST)
                     + b1_ref[...], 0.0)
    g2 = jnp.maximum(jnp.dot(g1, w2_ref[...],
                             preferred_element_type=jnp.float32,
                precision=lax.Precision.HIGHEST)
                     + b2_ref[...], 0.0)
    o_ref[...] = jnp.broadcast_to(g2, (8, U))


def _tc_proj(h, w1p, b1p, w2p, b2p):
    return pl.pallas_call(
        _proj_body,
        out_shape=jax.ShapeDtypeStruct((8, U), jnp.float32),
    )(h, w1p, b1p, w2p, b2p)


def _split_w(w, ind):
    blocks = w.reshape(9, ind // 9, -1)
    wa = jnp.concatenate([blocks[0], blocks[3], blocks[6]], axis=0)
    wb = jnp.concatenate([blocks[1] + blocks[2], blocks[4] + blocks[5],
                          blocks[7] + blocks[8]], axis=0)
    return wa, wb


def kernel(node_attributes, edge_indices, W0, b0, gamma0, beta0, mm0, mv0,
           W1, b1, gamma1, beta1, mm1, mv1, W2, b2, gamma2, beta2, mm2, mv2,
           Wp1, bp1, Wp2, bp2):
    dst = edge_indices[:, 0]
    src = edge_indices[:, 1]
    x = jnp.concatenate(
        [node_attributes, jnp.zeros((NP - N, D), jnp.float32)], axis=0)

    params = [(W0, b0, gamma0, beta0, mm0, mv0),
              (W1, b1, gamma1, beta1, mm1, mv1),
              (W2, b2, gamma2, beta2, mm2, mv2)]
    for i, (W, b, ga, be, mm, mv) in enumerate(params):
        wa, wb = _split_w(W, W.shape[0])
        ssum, smax, cnt = _sc_segment(x, dst, src)
        x = _tc_layer(ssum, smax, cnt[:, None], wa, wb,
                      b[None, :], mm[None, :], mv[None, :],
                      ga[None, :], be[None, :])

    w2p = jnp.concatenate([Wp2, jnp.zeros((U, U - Wp2.shape[1]),
                                          jnp.float32)], axis=1)
    b2p = jnp.concatenate([bp2, jnp.zeros((U - bp2.shape[0],),
                                          jnp.float32)])[None, :]
    out = _tc_proj(x, Wp1, bp1[None, :], w2p, b2p)
    g = out[0, :U // 2]
    return (g, jnp.stack([g, g], axis=0))


# speedup vs baseline: 2.9105x; 2.2862x over previous
import functools
import jax
import jax.numpy as jnp
from jax import lax
from jax.experimental import pallas as pl
from jax.experimental.pallas import tpu as pltpu
from jax.experimental.pallas import tpu_sc as plsc

N, E, D, U = 10000, 320000, 128, 128
NC, NS = 2, 16
NW = NC * NS
NP = 10240
R = NP // NW
DUMMY = R
AR = R + 16
S = 6400
NSLAB = E // S
CPS = S // 16
BG = 128
PB = 2 * BG
STAG = S + 16
EP = E + 8 * 1024

_mesh = plsc.VectorSubcoreMesh(core_axis_name="c", subcore_axis_name="s")
_params = pltpu.CompilerParams(needs_layout_passes=False)


@functools.partial(
    pl.kernel,
    out_type=(
        jax.ShapeDtypeStruct((NW * EP,), jnp.int32),
        jax.ShapeDtypeStruct((NW * EP,), jnp.int32),
        jax.ShapeDtypeStruct((NW * 16,), jnp.int32),
        jax.ShapeDtypeStruct((NP,), jnp.float32),
    ),
    mesh=_mesh,
    compiler_params=_params,
    scratch_types=[
        pltpu.VMEM((R,), jnp.float32),
        pltpu.VMEM((S,), jnp.int32),
        pltpu.VMEM((S,), jnp.int32),
        pltpu.VMEM((STAG,), jnp.int32),
        pltpu.VMEM((STAG,), jnp.int32),
        pltpu.VMEM((PB,), jnp.int32),
        pltpu.VMEM((PB,), jnp.int32),
        pltpu.VMEM((16,), jnp.int32),
    ],
)
def _sc_partition(dst_hbm, src_hbm, r_hbm, s_hbm, fill_hbm, cnt_hbm,
                  cntv, dstv, srcv, stag_r, stag_s, dpadr, dpads, fvec):
    wid = lax.axis_index("s") * NC + lax.axis_index("c")
    row_base = wid * R

    zeros16 = jnp.zeros((16,), jnp.float32)
    zi16 = jnp.zeros((16,), jnp.int32)
    dum16 = jnp.full((16,), DUMMY, jnp.int32)

    def init_cnt(i, _):
        cntv[pl.ds(i * 16, 16)] = zeros16
        return 0
    lax.fori_loop(0, R // 16, init_cnt, 0)

    def init_stag(i, _):
        stag_r[pl.ds(i * 16, 16)] = dum16
        stag_s[pl.ds(i * 16, 16)] = zi16
        return 0
    lax.fori_loop(0, STAG // 16, init_stag, 0)

    def init_pad(i, _):
        dpadr[pl.ds(i * 16, 16)] = dum16
        dpads[pl.ds(i * 16, 16)] = zi16
        return 0
    lax.fori_loop(0, PB // 16, init_pad, 0)

    ones16 = jnp.ones((16,), jnp.float32)

    def slab_body(si, total):
        pltpu.sync_copy(dst_hbm.at[pl.ds(si * S, S)], dstv)
        pltpu.sync_copy(src_hbm.at[pl.ds(si * S, S)], srcv)

        def chunk_body(c, fill):
            d16 = dstv[pl.ds(c * 16, 16)]
            s16 = srcv[pl.ds(c * 16, 16)]
            loc = d16 - row_base
            mask = jnp.logical_and(loc >= 0, loc < R)
            locc = jnp.where(mask, loc, 0)
            plsc.addupdate_scatter(cntv, [locc], ones16, mask=mask)
            plsc.store_compressed(stag_r.at[pl.ds(fill, 16)], loc, mask=mask)
            plsc.store_compressed(stag_s.at[pl.ds(fill, 16)], s16, mask=mask)
            return fill + plsc.all_reduce_population_count(mask)[0]

        fill = lax.fori_loop(0, CPS, chunk_body, 0)
        tmask = jnp.ones((16,), jnp.bool_)
        plsc.store_compressed(stag_r.at[pl.ds(fill, 16)], dum16, mask=tmask)
        plsc.store_compressed(stag_s.at[pl.ds(fill, 16)], zi16, mask=tmask)
        off = pl.multiple_of(wid * EP + total, 8)
        pltpu.sync_copy(stag_r, r_hbm.at[pl.ds(off, STAG)])
        pltpu.sync_copy(stag_s, s_hbm.at[pl.ds(off, STAG)])
        return total + ((fill + 15) // 16) * 16

    total = lax.fori_loop(0, NSLAB, slab_body, 0)
    off = pl.multiple_of(wid * EP + total, 8)
    pltpu.sync_copy(dpadr, r_hbm.at[pl.ds(off, PB)])
    pltpu.sync_copy(dpads, s_hbm.at[pl.ds(off, PB)])
    total = ((total + (PB - 1)) // PB) * PB

    fvec[pl.ds(0, 16)] = jnp.broadcast_to(total, (16,))
    pltpu.sync_copy(fvec, fill_hbm.at[pl.ds(wid * 16, 16)])
    pltpu.sync_copy(cntv, cnt_hbm.at[pl.ds(row_base, R)])


@functools.partial(
    pl.kernel,
    out_type=(
        jax.ShapeDtypeStruct((NP, D), jnp.float32),
        jax.ShapeDtypeStruct((NP, D), jnp.float32),
    ),
    mesh=_mesh,
    compiler_params=_params,
    scratch_types=[
        pltpu.VMEM((AR, D), jnp.float32),
        pltpu.VMEM((AR, D), jnp.float32),
        pltpu.VMEM((2, BG), jnp.int32),
        pltpu.VMEM((2, BG), jnp.int32),
        pltpu.VMEM((BG, D), jnp.float32),
        pltpu.VMEM((BG, D), jnp.float32),
        pltpu.VMEM((16,), jnp.int32),
        pltpu.SemaphoreType.DMA,
        pltpu.SemaphoreType.DMA,
    ],
)
def _sc_segment(x_hbm, r_hbm, s_hbm, fill_hbm, sum_hbm, max_hbm,
                accs, accm, sidx, ridx, gbuf0, gbuf1, fvec, sem0, sem1):
    wid = lax.axis_index("s") * NC + lax.axis_index("c")
    row_base = wid * R

    zeros16 = jnp.zeros((16,), jnp.float32)
    neginf16 = jnp.full((16,), -jnp.inf, jnp.float32)

    def init_row(r, _):
        for k in range(D // 16):
            accs[r, pl.ds(k * 16, 16)] = zeros16
            accm[r, pl.ds(k * 16, 16)] = neginf16
        return 0
    lax.fori_loop(0, AR, init_row, 0)

    pltpu.sync_copy(fill_hbm.at[pl.ds(wid * 16, 16)], fvec)
    fill = fvec[pl.ds(0, 16)][0]
    nb = fill // BG

    def fetch(b, p, gbuf, sem):
        base = jnp.minimum(b * BG, EP - BG)
        off = pl.multiple_of(wid * EP + base, 8)
        pltpu.sync_copy(s_hbm.at[pl.ds(off, BG)], sidx.at[p])
        pltpu.sync_copy(r_hbm.at[pl.ds(off, BG)], ridx.at[p])
        return pltpu.async_copy(x_hbm.at[sidx.at[p]], gbuf, sem)

    def process(p, gbuf):
        def group_body(jb, _):
            rv = ridx[p, pl.ds(jb * 16, 16)]
            for k in range(16):
                r = rv[k]
                j = jb * 16 + k
                for q in range(D // 16):
                    sl = pl.ds(q * 16, 16)
                    g = gbuf[j, sl]
                    accs[r, sl] = accs[r, sl] + g
                    accm[r, sl] = jnp.maximum(accm[r, sl], g)
            return 0
        lax.fori_loop(0, BG // 16, group_body, 0)

    @pl.when(nb > 0)
    def _():
        fetch(0, 0, gbuf0, sem0)
        fetch(1, 1, gbuf1, sem1)

    def drain(gbuf, sem):
        pltpu.make_async_copy(x_hbm.at[pl.ds(0, BG)], gbuf, sem).wait()

    def ring_body(gi, _):
        b0 = 2 * gi
        drain(gbuf0, sem0)
        process(0, gbuf0)

        @pl.when(b0 + 2 < nb)
        def _():
            fetch(b0 + 2, 0, gbuf0, sem0)

        drain(gbuf1, sem1)
        process(1, gbuf1)

        @pl.when(b0 + 3 < nb)
        def _():
            fetch(b0 + 3, 1, gbuf1, sem1)
        return 0

    lax.fori_loop(0, nb // 2, ring_body, 0)

    pltpu.sync_copy(accs.at[pl.ds(0, R)], sum_hbm.at[pl.ds(row_base, R)])
    pltpu.sync_copy(accm.at[pl.ds(0, R)], max_hbm.at[pl.ds(row_base, R)])


def _layer_body(s_ref, m_ref, c_ref, wa_ref, wb_ref, b_ref, mm_ref, mv_ref,
                ga_ref, be_ref, o_ref):
    s = s_ref[...]
    mx = m_ref[...]
    c = c_ref[...]
    deg = jnp.maximum(c, 1.0)
    inv = 1.0 / deg
    scale = jnp.log(deg + 1.0) * (1.0 / 2.302585092994046)
    mean = s * inv
    mxf = jnp.where(c == 0.0, 0.0, mx)
    feats = jnp.concatenate([mean, mxf, s], axis=1)
    a = jnp.dot(feats, wa_ref[...], preferred_element_type=jnp.float32,
                precision=lax.Precision.HIGHEST)
    b = jnp.dot(feats, wb_ref[...], preferred_element_type=jnp.float32,
                precision=lax.Precision.HIGHEST)
    h = jnp.maximum(a + scale * b + b_ref[...], 0.0)
    h = (h - mm_ref[...]) / jnp.sqrt(mv_ref[...] + 1e-3) * ga_ref[...] \
        + be_ref[...]
    o_ref[...] = h


_BR = 256


def _tc_layer(ssum, smax, cnt2d, wa, wb, b2, mm2, mv2, ga2, be2):
    full = lambda shp: pl.BlockSpec(shp, lambda i: (0, 0))
    return pl.pallas_call(
        _layer_body,
        grid=(NP // _BR,),
        in_specs=[
            pl.BlockSpec((_BR, D), lambda i: (i, 0)),
            pl.BlockSpec((_BR, D), lambda i: (i, 0)),
            pl.BlockSpec((_BR, 1), lambda i: (i, 0)),
            full((3 * D, U)), full((3 * D, U)),
            full((1, U)), full((1, U)), full((1, U)), full((1, U)),
            full((1, U)),
        ],
        out_specs=pl.BlockSpec((_BR, U), lambda i: (i, 0)),
        out_shape=jax.ShapeDtypeStruct((NP, U), jnp.float32),
    )(ssum, smax, cnt2d, wa, wb, b2, mm2, mv2, ga2, be2)


def _proj_body(h_ref, w1_ref, b1_ref, w2_ref, b2_ref, o_ref):
    h = h_ref[...]
    rid = lax.broadcasted_iota(jnp.int32, (NP, U), 0)
    g = jnp.sum(jnp.where(rid < N, h, 0.0), axis=0, keepdims=True) / N
    g1 = jnp.maximum(jnp.dot(g, w1_ref[...],
                             preferred_element_type=jnp.float32,
                             precision=lax.Precision.HIGHEST)
                     + b1_ref[...], 0.0)
    g2 = jnp.maximum(jnp.dot(g1, w2_ref[...],
                             preferred_element_type=jnp.float32,
                             precision=lax.Precision.HIGHEST)
                     + b2_ref[...], 0.0)
    o_ref[...] = jnp.broadcast_to(g2, (8, U))


def _tc_proj(h, w1p, b1p, w2p, b2p):
    return pl.pallas_call(
        _proj_body,
        out_shape=jax.ShapeDtypeStruct((8, U), jnp.float32),
    )(h, w1p, b1p, w2p, b2p)


def _split_w(w, ind):
    blocks = w.reshape(9, ind // 9, -1)
    wa = jnp.concatenate([blocks[0], blocks[3], blocks[6]], axis=0)
    wb = jnp.concatenate([blocks[1] + blocks[2], blocks[4] + blocks[5],
                          blocks[7] + blocks[8]], axis=0)
    return wa, wb


def kernel(node_attributes, edge_indices, W0, b0, gamma0, beta0, mm0, mv0,
           W1, b1, gamma1, beta1, mm1, mv1, W2, b2, gamma2, beta2, mm2, mv2,
           Wp1, bp1, Wp2, bp2):
    dst = edge_indices[:, 0]
    src = edge_indices[:, 1]
    x = jnp.concatenate(
        [node_attributes, jnp.zeros((NP - N, D), jnp.float32)], axis=0)

    r_part, s_part, fills, cnt = _sc_partition(dst, src)

    params = [(W0, b0, gamma0, beta0, mm0, mv0),
              (W1, b1, gamma1, beta1, mm1, mv1),
              (W2, b2, gamma2, beta2, mm2, mv2)]
    for W, b, ga, be, mm, mv in params:
        wa, wb = _split_w(W, W.shape[0])
        ssum, smax = _sc_segment(x, r_part, s_part, fills)
        x = _tc_layer(ssum, smax, cnt[:, None], wa, wb,
                      b[None, :], mm[None, :], mv[None, :],
                      ga[None, :], be[None, :])

    w2p = jnp.concatenate([Wp2, jnp.zeros((U, U - Wp2.shape[1]),
                                          jnp.float32)], axis=1)
    b2p = jnp.concatenate([bp2, jnp.zeros((U - bp2.shape[0],),
                                          jnp.float32)])[None, :]
    out = _tc_proj(x, Wp1, bp1[None, :], w2p, b2p)
    g = out[0, :U // 2]
    return (g, jnp.stack([g, g], axis=0))
